# baseline (speedup 1.0000x reference)
import jax
import jax.numpy as jnp
from jax import lax
from jax.experimental import pallas as pl

_C = 32
_GAMMA = 12.0
_CLIP = 0.01
_BN = 8


def _routing_body(x_ref, bw_ref, mask_ref, ranks_ref):
    m1 = x_ref[:, pl.Slice(0, 64, 8), :]
    for k in range(1, 8):
        m1 = jnp.maximum(m1, x_ref[:, pl.Slice(k, 64, 8), :])
    t = jnp.maximum(m1[:, :, 0:64], m1[:, :, 64:128])
    m = jnp.maximum(t[:, :, 0:32], t[:, :, 32:64])
    s = m.sum(axis=1)
    s = s * bw_ref[...]

    vk = s[:, :, None]
    vc = s[:, None, :]
    k_idx = lax.broadcasted_iota(jnp.int32, (_BN, _C, _C), 1)
    c_idx = lax.broadcasted_iota(jnp.int32, (_BN, _C, _C), 2)
    cmp = (vk > vc) | ((vk == vc) & (k_idx < c_idx))
    ranks = cmp.astype(jnp.int32).sum(axis=1)

    mask = jnp.exp(ranks.astype(jnp.float32) * (-_GAMMA / (_C - 1)))
    mask = jnp.where(mask < _CLIP, 0.0, mask)
    mask_ref[...] = mask
    ranks_ref[...] = ranks


def kernel(routings, boosting_weights):
    n = routings.shape[0]
    x = routings.reshape(n, 512, 128)
    bw = boosting_weights.reshape(1, _C)
    mask, ranks = pl.pallas_call(
        _routing_body,
        grid=(n // _BN,),
        in_specs=[
            pl.BlockSpec((_BN, 512, 128), lambda i: (i, 0, 0)),
            pl.BlockSpec((1, _C), lambda i: (0, 0)),
        ],
        out_specs=[
            pl.BlockSpec((_BN, _C), lambda i: (i, 0)),
            pl.BlockSpec((_BN, _C), lambda i: (i, 0)),
        ],
        out_shape=[
            jax.ShapeDtypeStruct((n, _C), jnp.float32),
            jax.ShapeDtypeStruct((n, _C), jnp.int32),
        ],
    )(x, bw)
    return mask, ranks
